# baseline (speedup 1.0000x reference)
import functools

import jax
import jax.numpy as jnp
from jax import lax
from jax.experimental import pallas as pl
from jax.experimental.pallas import tpu as pltpu
from jax.experimental.pallas import tpu_sc as plsc

_NC = 2
_NS = 16
_NW = _NC * _NS
_W = 128


def _sc_gather(table, xf):
    (v,) = table.shape
    (n,) = xf.shape
    per_w = n // _NW
    chunks = per_w // _W
    t_sub = 15632
    mesh = plsc.VectorSubcoreMesh(core_axis_name="c", subcore_axis_name="s")

    @functools.partial(
        pl.kernel,
        mesh=mesh,
        out_type=jax.ShapeDtypeStruct((n,), jnp.int32),
        scratch_types=[
            pltpu.VMEM((per_w,), jnp.int32),
            pltpu.VMEM((per_w,), jnp.int32),
            pltpu.VMEM((t_sub,), jnp.int32),
            pltpu.VMEM_SHARED((v,), jnp.int32),
            pltpu.SemaphoreType.DMA,
        ],
    )
    def body(table_hbm, xf_hbm, out_hbm, idx_v, out_v, tab_v, tab_s, sem):
        sid = lax.axis_index("s")
        wid = sid * _NC + lax.axis_index("c")
        base = wid * per_w

        @pl.loop(0, 4)
        def _stage(k):
            start = jnp.minimum((sid * 4 + k) * t_sub, v - t_sub)
            pltpu.sync_copy(table_hbm.at[pl.ds(start, t_sub)], tab_v)
            pltpu.sync_copy(tab_v, tab_s.at[pl.ds(start, t_sub)])

        pltpu.sync_copy(xf_hbm.at[pl.ds(base, per_w)], idx_v)
        plsc.subcore_barrier()

        @pl.loop(0, chunks)
        def _fire(j):
            pltpu.async_copy(tab_s.at[idx_v.at[pl.ds(j * _W, _W)]],
                             out_v.at[pl.ds(j * _W, _W)], sem)

        pltpu.make_async_copy(out_hbm.at[pl.ds(base, per_w)], out_v, sem).wait()
        pltpu.sync_copy(out_v, out_hbm.at[pl.ds(base, per_w)])

    return body(table, xf)


def kernel(domain_mapping, x):
    b, f = x.shape
    nch = 2
    bc = b // nch
    outs = []
    for k in range(nch):
        xk = lax.slice_in_dim(x, k * bc, (k + 1) * bc, axis=0)
        ok = _sc_gather(domain_mapping, xk.reshape(bc * f))
        outs.append(ok.reshape(bc, f))
    return jnp.concatenate(outs, axis=0)


# speedup vs baseline: 1.0678x; 1.0678x over previous
import functools

import jax
import jax.numpy as jnp
from jax import lax
from jax.experimental import pallas as pl
from jax.experimental.pallas import tpu as pltpu
from jax.experimental.pallas import tpu_sc as plsc

_NC = 2
_NS = 16
_NW = _NC * _NS
_W = 128


def _sc_gather(table, xf):
    (n,) = xf.shape
    per_w = n // _NW
    chunks = per_w // _W
    mesh = plsc.VectorSubcoreMesh(core_axis_name="c", subcore_axis_name="s")

    @functools.partial(
        pl.kernel,
        mesh=mesh,
        out_type=jax.ShapeDtypeStruct((n,), jnp.int32),
        scratch_types=[
            pltpu.VMEM((per_w,), jnp.int32),
            pltpu.VMEM((per_w,), jnp.int32),
            pltpu.SemaphoreType.DMA,
        ],
    )
    def body(table_hbm, xf_hbm, out_hbm, idx_v, out_v, sem):
        wid = lax.axis_index("s") * _NC + lax.axis_index("c")
        base = wid * per_w
        pltpu.sync_copy(xf_hbm.at[pl.ds(base, per_w)], idx_v)

        @pl.loop(0, chunks)
        def _fire(j):
            pltpu.async_copy(table_hbm.at[idx_v.at[pl.ds(j * _W, _W)]],
                             out_v.at[pl.ds(j * _W, _W)], sem)

        pltpu.make_async_copy(out_hbm.at[pl.ds(base, per_w)], out_v, sem).wait()
        pltpu.sync_copy(out_v, out_hbm.at[pl.ds(base, per_w)])

    return body(table, xf)


def kernel(domain_mapping, x):
    b, f = x.shape
    nch = 2
    bc = b // nch
    outs = []
    for k in range(nch):
        xk = lax.slice_in_dim(x, k * bc, (k + 1) * bc, axis=0)
        ok = _sc_gather(domain_mapping, xk.reshape(bc * f))
        outs.append(ok.reshape(bc, f))
    return jnp.concatenate(outs, axis=0)
